# Initial kernel scaffold; baseline (speedup 1.0000x reference)
import jax
import jax.numpy as jnp
from jax.experimental import pallas as pl


def kernel(node_attr, edge_index, edge_attr, edge_sh, W1, b1, W2, b2):
    raise NotImplementedError("write your pallas kernel here")


# speedup vs baseline: 1.6903x; 1.6903x over previous
import functools

import jax
import jax.numpy as jnp
import numpy as np
from jax.experimental import pallas as pl
from jax.experimental.pallas import tpu as pltpu

_NS = 16
_NV = 8
_DIN = _NS + 3 * _NV
_W0IN = _NS + _NV
_W0NUM = _W0IN * _NS
_W1NUM = _W0IN * _NV
_WNUM = _W0NUM + _W1NUM
_NNODES = 10000

_PERM = np.concatenate([
    np.arange(_NS),
    *[np.array([_NS + 3 * j + c for j in range(_NV)]) for c in range(3)],
]).astype(np.int32)
_INV_PERM = np.argsort(_PERM).astype(np.int32)

def _expand_mat(n_in, n_out, scale_rows=None):
    r = np.zeros((n_in, n_in * n_out), np.float32)
    for i in range(n_in):
        s = 1.0 if scale_rows is None else scale_rows[i]
        r[i, i * n_out:(i + 1) * n_out] = s
    return r

def _sum_mat(n_in, n_out, scale):
    s = np.zeros((n_in * n_out, n_out), np.float32)
    for i in range(n_in):
        s[i * n_out:(i + 1) * n_out, :] = np.eye(n_out, dtype=np.float32) * scale
    return s

_R16 = _expand_mat(_W0IN, _NS, np.array([1.0] * _NS + [1.0 / np.sqrt(3.0)] * _NV))
_S16 = _sum_mat(_W0IN, _NS, 1.0 / np.sqrt(_W0IN))
_R8 = _expand_mat(_W0IN, _NV)
_S8 = _sum_mat(_W0IN, _NV, 1.0 / np.sqrt(_W0IN))

_TILE = 1000


def _edge_kernel(x_ref, ea_ref, sh_ref, w1_ref, b1_ref, w2_ref, b2_ref,
                 r16_ref, s16_ref, r8_ref, s8_ref, out_ref):
    ea = ea_ref[...]
    x = x_ref[...]
    sh = sh_ref[...]
    h = jnp.maximum(ea @ w1_ref[...] + b1_ref[...], 0.0)
    tpw = h @ w2_ref[...] + b2_ref[...]
    tpw0 = tpw[:, :_W0NUM]
    tpw1 = tpw[:, _W0NUM:]
    in0e = x[:, :_NS]
    v = [x[:, _NS + c * _NV:_NS + (c + 1) * _NV] for c in range(3)]
    sh0 = sh[:, 0:1]
    s123 = [sh[:, c + 1:c + 2] for c in range(3)]
    o0e = jnp.concatenate(
        [in0e * sh0, v[0] * s123[0] + v[1] * s123[1] + v[2] * s123[2]], axis=1)
    out0 = ((o0e @ r16_ref[...]) * tpw0) @ s16_ref[...]
    outs = [out0]
    for c in range(3):
        o1o_c = jnp.concatenate([in0e * s123[c], v[c] * sh0], axis=1)
        outs.append(((o1o_c @ r8_ref[...]) * tpw1) @ s8_ref[...])
    out_ref[...] = jnp.concatenate(outs, axis=1)


def _edge_messages(x, edge_attr, edge_sh, W1, b1, W2, b2):
    E = edge_attr.shape[0]
    grid = E // _TILE
    spec = lambda cols: pl.BlockSpec((_TILE, cols), lambda i: (i, 0))
    const = lambda shape: pl.BlockSpec(shape, lambda i: (0, 0))
    return pl.pallas_call(
        _edge_kernel,
        grid=(grid,),
        in_specs=[
            spec(_DIN), spec(16), spec(4),
            const((16, 64)), const((1, 64)), const((64, _WNUM)),
            const((1, _WNUM)),
            const(_R16.shape), const(_S16.shape), const(_R8.shape),
            const(_S8.shape),
        ],
        out_specs=spec(_DIN),
        out_shape=jax.ShapeDtypeStruct((E, _DIN), jnp.float32),
        compiler_params=pltpu.CompilerParams(
            dimension_semantics=("parallel",)),
    )(x, edge_attr, edge_sh, W1, b1.reshape(1, -1), W2, b2.reshape(1, -1),
      jnp.asarray(_R16), jnp.asarray(_S16), jnp.asarray(_R8), jnp.asarray(_S8))


def kernel(node_attr, edge_index, edge_attr, edge_sh, W1, b1, W2, b2):
    edge_src = edge_index[0]
    edge_dst = edge_index[1]
    node_p = node_attr[:, _PERM]
    x = node_p[edge_dst]
    msg = _edge_messages(x, edge_attr, edge_sh, W1, b1, W2, b2)
    sums = jax.ops.segment_sum(msg, edge_src, num_segments=_NNODES)
    counts = jax.ops.segment_sum(jnp.ones((edge_src.shape[0],), jnp.float32),
                                 edge_src, num_segments=_NNODES)
    out_p = sums / jnp.clip(counts, 1.0)[:, None] + node_p
    return out_p[:, _INV_PERM]


# speedup vs baseline: 2.1570x; 1.2761x over previous
import functools

import jax
import jax.numpy as jnp
import numpy as np
from jax import lax
from jax.experimental import pallas as pl
from jax.experimental.pallas import tpu as pltpu
from jax.experimental.pallas import tpu_sc as plsc

_NS = 16
_NV = 8
_DIN = _NS + 3 * _NV
_W0IN = _NS + _NV
_W0NUM = _W0IN * _NS
_W1NUM = _W0IN * _NV
_WNUM = _W0NUM + _W1NUM
_NNODES = 10000
_E = 160000

_MSG_W = 48
_NROWS = _NNODES + 112
_NW = 32
_CHUNK = 128
_NCHUNK = 40
_EPW = _CHUNK * _NCHUNK
_EP = _NW * _EPW
_TILE = 1024
_RPW = _NROWS // 16

_PERM = np.concatenate([
    np.arange(_NS),
    *[np.array([_NS + 3 * j + c for j in range(_NV)]) for c in range(3)],
]).astype(np.int32)
_PINV = np.zeros((_DIN, _DIN), np.float32)
for _j in range(_DIN):
    _PINV[_j, _PERM[_j]] = 1.0


def _expand_mat(n_in, n_out, scale_rows=None):
    r = np.zeros((n_in, n_in * n_out), np.float32)
    for i in range(n_in):
        s = 1.0 if scale_rows is None else scale_rows[i]
        r[i, i * n_out:(i + 1) * n_out] = s
    return r


def _sum_mat(n_in, n_out, scale):
    s = np.zeros((n_in * n_out, n_out), np.float32)
    for i in range(n_in):
        s[i * n_out:(i + 1) * n_out, :] = np.eye(n_out, dtype=np.float32) * scale
    return s


_R16 = _expand_mat(_W0IN, _NS, np.array([1.0] * _NS + [1.0 / np.sqrt(3.0)] * _NV))
_S16 = _sum_mat(_W0IN, _NS, 1.0 / np.sqrt(_W0IN))
_R8 = _expand_mat(_W0IN, _NV)
_S8 = _sum_mat(_W0IN, _NV, 1.0 / np.sqrt(_W0IN))


def _edge_kernel(x_ref, ea_ref, sh_ref, w1_ref, b1_ref, w2_ref, b2_ref,
                 r16_ref, s16_ref, r8_ref, s8_ref, out_ref):
    ea = ea_ref[...]
    x = x_ref[...]
    sh = sh_ref[...]
    h = jnp.maximum(ea @ w1_ref[...] + b1_ref[...], 0.0)
    tpw = h @ w2_ref[...] + b2_ref[...]
    tpw0 = tpw[:, :_W0NUM]
    tpw1 = tpw[:, _W0NUM:]
    in0e = x[:, :_NS]
    v = [x[:, _NS + c * _NV:_NS + (c + 1) * _NV] for c in range(3)]
    sh0 = sh[:, 0:1]
    s123 = [sh[:, c + 1:c + 2] for c in range(3)]
    o0e = jnp.concatenate(
        [in0e * sh0, v[0] * s123[0] + v[1] * s123[1] + v[2] * s123[2]], axis=1)
    outs = [((o0e @ r16_ref[...]) * tpw0) @ s16_ref[...]]
    for c in range(3):
        o1o_c = jnp.concatenate([in0e * s123[c], v[c] * sh0], axis=1)
        outs.append(((o1o_c @ r8_ref[...]) * tpw1) @ s8_ref[...])
    t = out_ref.shape[0]
    outs.append(jnp.ones((t, 1), jnp.float32))
    outs.append(jnp.zeros((t, 7), jnp.float32))
    out_ref[...] = jnp.concatenate(outs, axis=1)


def _edge_messages(x, edge_attr, edge_sh, W1, b1, W2, b2):
    grid = _EP // _TILE
    spec = lambda cols: pl.BlockSpec((_TILE, cols), lambda i: (i, 0))
    const = lambda shape: pl.BlockSpec(shape, lambda i: (0, 0))
    return pl.pallas_call(
        _edge_kernel,
        grid=(grid,),
        in_specs=[
            spec(_DIN), spec(16), spec(4),
            const((16, 64)), const((1, 64)), const((64, _WNUM)),
            const((1, _WNUM)),
            const(_R16.shape), const(_S16.shape), const(_R8.shape),
            const(_S8.shape),
        ],
        out_specs=spec(_MSG_W),
        out_shape=jax.ShapeDtypeStruct((_EP, _MSG_W), jnp.float32),
        compiler_params=pltpu.CompilerParams(
            dimension_semantics=("parallel",)),
    )(x, edge_attr, edge_sh, W1, b1.reshape(1, -1), W2, b2.reshape(1, -1),
      jnp.asarray(_R16), jnp.asarray(_S16), jnp.asarray(_R8), jnp.asarray(_S8))


def _scatter_partials(msg, src_idx3, zero_rows):
    mesh = plsc.VectorSubcoreMesh(core_axis_name="c", subcore_axis_name="s")

    @functools.partial(
        pl.kernel, mesh=mesh,
        out_type=jax.ShapeDtypeStruct((2, _NROWS, _MSG_W), jnp.float32),
        scratch_types=[
            pltpu.VMEM_SHARED((_NROWS, _MSG_W), jnp.float32),
            pltpu.VMEM((_NCHUNK, _CHUNK), jnp.int32),
            pltpu.VMEM((_CHUNK, _MSG_W), jnp.float32),
        ],
    )
    def k(msg_hbm, idx_hbm, zero_hbm, out_hbm, acc, idx_v, msg_v):
        c = lax.axis_index("c")
        s = lax.axis_index("s")
        w = c * 16 + s
        roff = s * _RPW
        pltpu.sync_copy(zero_hbm.at[pl.ds(roff, _RPW)], acc.at[pl.ds(roff, _RPW)])
        pltpu.sync_copy(idx_hbm.at[w], idx_v)
        plsc.subcore_barrier()

        def body(j, carry):
            base = w * _EPW + j * _CHUNK
            pltpu.sync_copy(msg_hbm.at[pl.ds(base, _CHUNK)], msg_v)
            pltpu.sync_copy(msg_v, acc.at[idx_v.at[j]], add=True)
            return carry

        lax.fori_loop(0, _NCHUNK, body, 0)
        plsc.subcore_barrier()
        pltpu.sync_copy(acc.at[pl.ds(roff, _RPW)],
                        out_hbm.at[c, pl.ds(roff, _RPW)])

    return k(msg, src_idx3, zero_rows)


def _epilogue_kernel(p_ref, na_ref, pinv_ref, out_ref):
    p = p_ref[0] + p_ref[1]
    cnt = p[:, _DIN:_DIN + 1]
    vals = p[:, :_DIN]
    out_ref[...] = (vals / jnp.maximum(cnt, 1.0)) @ pinv_ref[...] + na_ref[...]


def _epilogue(partials, node_attr, pinv):
    rows = 1256
    return pl.pallas_call(
        _epilogue_kernel,
        grid=(pl.cdiv(_NNODES, rows),),
        in_specs=[
            pl.BlockSpec((2, rows, _MSG_W), lambda i: (0, i, 0)),
            pl.BlockSpec((rows, _DIN), lambda i: (i, 0)),
            pl.BlockSpec((_DIN, _DIN), lambda i: (0, 0)),
        ],
        out_specs=pl.BlockSpec((rows, _DIN), lambda i: (i, 0)),
        out_shape=jax.ShapeDtypeStruct((_NNODES, _DIN), jnp.float32),
        compiler_params=pltpu.CompilerParams(
            dimension_semantics=("parallel",)),
    )(partials, node_attr, pinv)


def kernel(node_attr, edge_index, edge_attr, edge_sh, W1, b1, W2, b2):
    edge_src = edge_index[0]
    edge_dst = edge_index[1]
    pad = _EP - _E
    node_p = node_attr[:, _PERM]
    dst_pad = jnp.concatenate([edge_dst, jnp.zeros((pad,), jnp.int32)])
    src_pad = jnp.concatenate(
        [edge_src, jnp.full((pad,), _NNODES, jnp.int32)])
    ea_pad = jnp.pad(edge_attr, ((0, pad), (0, 0)))
    sh_pad = jnp.pad(edge_sh, ((0, pad), (0, 0)))
    x = node_p[dst_pad]
    msg = _edge_messages(x, ea_pad, sh_pad, W1, b1, W2, b2)
    partials = _scatter_partials(
        msg, src_pad.reshape(_NW, _NCHUNK, _CHUNK),
        jnp.zeros((_NROWS, _MSG_W), jnp.float32))
    return _epilogue(partials, node_attr, jnp.asarray(_PINV))


# speedup vs baseline: 2.9725x; 1.3781x over previous
import functools

import jax
import jax.numpy as jnp
import numpy as np
from jax import lax
from jax.experimental import pallas as pl
from jax.experimental.pallas import tpu as pltpu
from jax.experimental.pallas import tpu_sc as plsc

_NS = 16
_NV = 8
_DIN = _NS + 3 * _NV
_W0IN = _NS + _NV
_W0NUM = _W0IN * _NS
_W1NUM = _W0IN * _NV
_WNUM = _W0NUM + _W1NUM
_NNODES = 10000
_E = 160000

_MSG_W = 128
_NROWS = _NNODES + 112
_NW = 32
_CHUNK = 128
_NCHUNK = 40
_EPW = _CHUNK * _NCHUNK
_EP = _NW * _EPW
_TILE = 1024
_RPW = _NROWS // 16

_PERM = np.concatenate([
    np.arange(_NS),
    *[np.array([_NS + 3 * j + c for j in range(_NV)]) for c in range(3)],
]).astype(np.int32)
_PINV = np.zeros((_DIN, _DIN), np.float32)
for _j in range(_DIN):
    _PINV[_j, _PERM[_j]] = 1.0


def _expand_mat(n_in, n_out, scale_rows=None):
    r = np.zeros((n_in, n_in * n_out), np.float32)
    for i in range(n_in):
        s = 1.0 if scale_rows is None else scale_rows[i]
        r[i, i * n_out:(i + 1) * n_out] = s
    return r


def _sum_mat(n_in, n_out, scale):
    s = np.zeros((n_in * n_out, n_out), np.float32)
    for i in range(n_in):
        s[i * n_out:(i + 1) * n_out, :] = np.eye(n_out, dtype=np.float32) * scale
    return s


_R16 = _expand_mat(_W0IN, _NS, np.array([1.0] * _NS + [1.0 / np.sqrt(3.0)] * _NV))
_S16 = _sum_mat(_W0IN, _NS, 1.0 / np.sqrt(_W0IN))
_R8 = _expand_mat(_W0IN, _NV)
_S8 = _sum_mat(_W0IN, _NV, 1.0 / np.sqrt(_W0IN))


def _edge_kernel(x_ref, ea_ref, sh_ref, w1_ref, b1_ref, w2_ref, b2_ref,
                 r16_ref, s16_ref, r8_ref, s8_ref, out_ref):
    ea = ea_ref[...]
    x = x_ref[:, :_DIN]
    sh = sh_ref[...]
    h = jnp.maximum(ea @ w1_ref[...] + b1_ref[...], 0.0)
    tpw = h @ w2_ref[...] + b2_ref[...]
    tpw0 = tpw[:, :_W0NUM]
    tpw1 = tpw[:, _W0NUM:]
    in0e = x[:, :_NS]
    v = [x[:, _NS + c * _NV:_NS + (c + 1) * _NV] for c in range(3)]
    sh0 = sh[:, 0:1]
    s123 = [sh[:, c + 1:c + 2] for c in range(3)]
    o0e = jnp.concatenate(
        [in0e * sh0, v[0] * s123[0] + v[1] * s123[1] + v[2] * s123[2]], axis=1)
    outs = [((o0e @ r16_ref[...]) * tpw0) @ s16_ref[...]]
    for c in range(3):
        o1o_c = jnp.concatenate([in0e * s123[c], v[c] * sh0], axis=1)
        outs.append(((o1o_c @ r8_ref[...]) * tpw1) @ s8_ref[...])
    t = out_ref.shape[0]
    outs.append(jnp.ones((t, 1), jnp.float32))
    outs.append(jnp.zeros((t, _MSG_W - _DIN - 1), jnp.float32))
    out_ref[...] = jnp.concatenate(outs, axis=1)


def _edge_messages(x, edge_attr, edge_sh, W1, b1, W2, b2):
    grid = _EP // _TILE
    spec = lambda cols: pl.BlockSpec((_TILE, cols), lambda i: (i, 0))
    const = lambda shape: pl.BlockSpec(shape, lambda i: (0, 0))
    return pl.pallas_call(
        _edge_kernel,
        grid=(grid,),
        in_specs=[
            spec(128), spec(16), spec(4),
            const((16, 64)), const((1, 64)), const((64, _WNUM)),
            const((1, _WNUM)),
            const(_R16.shape), const(_S16.shape), const(_R8.shape),
            const(_S8.shape),
        ],
        out_specs=spec(_MSG_W),
        out_shape=jax.ShapeDtypeStruct((_EP, _MSG_W), jnp.float32),
        compiler_params=pltpu.CompilerParams(
            dimension_semantics=("parallel",)),
    )(x, edge_attr, edge_sh, W1, b1.reshape(1, -1), W2, b2.reshape(1, -1),
      jnp.asarray(_R16), jnp.asarray(_S16), jnp.asarray(_R8), jnp.asarray(_S8))


def _gather_rows(table, dst_idx3):
    mesh = plsc.VectorSubcoreMesh(core_axis_name="c", subcore_axis_name="s")

    @functools.partial(
        pl.kernel, mesh=mesh,
        out_type=jax.ShapeDtypeStruct((_EP, 128), jnp.float32),
        scratch_types=[
            pltpu.VMEM((_CHUNK,), jnp.int32),
            pltpu.VMEM((_CHUNK, 128), jnp.float32),
            pltpu.SemaphoreType.DMA,
        ],
    )
    def k(table_hbm, idx_hbm, out_hbm, idx_c, rows_v, sem):
        c = lax.axis_index("c")
        s = lax.axis_index("s")
        w = c * 16 + s

        def body(j, carry):
            base = w * _EPW + j * _CHUNK
            pltpu.sync_copy(idx_hbm.at[w, j], idx_c)
            pltpu.async_copy(table_hbm.at[idx_c], rows_v, sem).wait()
            pltpu.sync_copy(rows_v, out_hbm.at[pl.ds(base, _CHUNK)])
            return carry

        lax.fori_loop(0, _NCHUNK, body, 0)

    return k(table, dst_idx3)


def _scatter_partials(msg, src_idx3, zero_rows):
    mesh = plsc.VectorSubcoreMesh(core_axis_name="c", subcore_axis_name="s")

    @functools.partial(
        pl.kernel, mesh=mesh,
        out_type=jax.ShapeDtypeStruct((2, _NROWS, _MSG_W), jnp.float32),
        scratch_types=[
            pltpu.VMEM_SHARED((_NROWS, _MSG_W), jnp.float32),
            pltpu.VMEM((_NCHUNK, _CHUNK), jnp.int32),
            pltpu.VMEM((_CHUNK, _MSG_W), jnp.float32),
        ],
    )
    def k(msg_hbm, idx_hbm, zero_hbm, out_hbm, acc, idx_v, msg_v):
        c = lax.axis_index("c")
        s = lax.axis_index("s")
        w = c * 16 + s
        roff = s * _RPW
        pltpu.sync_copy(zero_hbm.at[pl.ds(roff, _RPW)], acc.at[pl.ds(roff, _RPW)])
        pltpu.sync_copy(idx_hbm.at[w], idx_v)
        plsc.subcore_barrier()

        def body(j, carry):
            base = w * _EPW + j * _CHUNK
            pltpu.sync_copy(msg_hbm.at[pl.ds(base, _CHUNK)], msg_v)
            pltpu.sync_copy(msg_v, acc.at[idx_v.at[j]], add=True)
            return carry

        lax.fori_loop(0, _NCHUNK, body, 0)
        plsc.subcore_barrier()
        pltpu.sync_copy(acc.at[pl.ds(roff, _RPW)],
                        out_hbm.at[c, pl.ds(roff, _RPW)])

    return k(msg, src_idx3, zero_rows)


def _epilogue_kernel(p_ref, na_ref, pinv_ref, out_ref):
    p = p_ref[0] + p_ref[1]
    cnt = p[:, _DIN:_DIN + 1]
    vals = p[:, :_DIN]
    out_ref[...] = (vals / jnp.maximum(cnt, 1.0)) @ pinv_ref[...] + na_ref[...]


def _epilogue(partials, node_attr, pinv):
    rows = 1256
    return pl.pallas_call(
        _epilogue_kernel,
        grid=(pl.cdiv(_NNODES, rows),),
        in_specs=[
            pl.BlockSpec((2, rows, _MSG_W), lambda i: (0, i, 0)),
            pl.BlockSpec((rows, _DIN), lambda i: (i, 0)),
            pl.BlockSpec((_DIN, _DIN), lambda i: (0, 0)),
        ],
        out_specs=pl.BlockSpec((rows, _DIN), lambda i: (i, 0)),
        out_shape=jax.ShapeDtypeStruct((_NNODES, _DIN), jnp.float32),
        compiler_params=pltpu.CompilerParams(
            dimension_semantics=("parallel",)),
    )(partials, node_attr, pinv)


def kernel(node_attr, edge_index, edge_attr, edge_sh, W1, b1, W2, b2):
    edge_src = edge_index[0]
    edge_dst = edge_index[1]
    pad = _EP - _E
    node_p = jnp.pad(node_attr[:, _PERM], ((0, 0), (0, 128 - _DIN)))
    dst_pad = jnp.concatenate([edge_dst, jnp.zeros((pad,), jnp.int32)])
    src_pad = jnp.concatenate(
        [edge_src, jnp.full((pad,), _NNODES, jnp.int32)])
    ea_pad = jnp.pad(edge_attr, ((0, pad), (0, 0)))
    sh_pad = jnp.pad(edge_sh, ((0, pad), (0, 0)))
    x = _gather_rows(node_p, dst_pad.reshape(_NW, _NCHUNK, _CHUNK))
    msg = _edge_messages(x, ea_pad, sh_pad, W1, b1, W2, b2)
    partials = _scatter_partials(
        msg, src_pad.reshape(_NW, _NCHUNK, _CHUNK),
        jnp.zeros((_NROWS, _MSG_W), jnp.float32))
    return _epilogue(partials, node_attr, jnp.asarray(_PINV))


# speedup vs baseline: 3.0672x; 1.0319x over previous
import functools

import jax
import jax.numpy as jnp
import numpy as np
from jax import lax
from jax.experimental import pallas as pl
from jax.experimental.pallas import tpu as pltpu
from jax.experimental.pallas import tpu_sc as plsc

_NS = 16
_NV = 8
_DIN = _NS + 3 * _NV
_W0IN = _NS + _NV
_W0NUM = _W0IN * _NS
_W1NUM = _W0IN * _NV
_WNUM = _W0NUM + _W1NUM
_NNODES = 10000
_E = 160000

_MSG_W = 128
_NROWS = _NNODES + 112
_NW = 32
_CHUNK = 128
_NCHUNK = 40
_EPW = _CHUNK * _NCHUNK
_EP = _NW * _EPW
_TILE = 1024
_RPW = _NROWS // 16

_PERM = np.concatenate([
    np.arange(_NS),
    *[np.array([_NS + 3 * j + c for j in range(_NV)]) for c in range(3)],
]).astype(np.int32)
_PINV = np.zeros((_DIN, _DIN), np.float32)
for _j in range(_DIN):
    _PINV[_j, _PERM[_j]] = 1.0


def _expand_mat(n_in, n_out, scale_rows=None):
    r = np.zeros((n_in, n_in * n_out), np.float32)
    for i in range(n_in):
        s = 1.0 if scale_rows is None else scale_rows[i]
        r[i, i * n_out:(i + 1) * n_out] = s
    return r


def _sum_mat(n_in, n_out, scale):
    s = np.zeros((n_in * n_out, n_out), np.float32)
    for i in range(n_in):
        s[i * n_out:(i + 1) * n_out, :] = np.eye(n_out, dtype=np.float32) * scale
    return s


_R16 = _expand_mat(_W0IN, _NS, np.array([1.0] * _NS + [1.0 / np.sqrt(3.0)] * _NV))
_S16 = _sum_mat(_W0IN, _NS, 1.0 / np.sqrt(_W0IN))
_R8 = _expand_mat(_W0IN, _NV)
_S8 = _sum_mat(_W0IN, _NV, 1.0 / np.sqrt(_W0IN))


def _edge_kernel(x_ref, ea_ref, sh_ref, w1_ref, b1_ref, w2_ref, b2_ref,
                 r16_ref, s16_ref, r8_ref, s8_ref, out_ref):
    ea = ea_ref[...]
    x = x_ref[:, :_DIN]
    sh = sh_ref[...]
    h = jnp.maximum(ea @ w1_ref[...] + b1_ref[...], 0.0)
    tpw = h @ w2_ref[...] + b2_ref[...]
    tpw0 = tpw[:, :_W0NUM]
    tpw1 = tpw[:, _W0NUM:]
    in0e = x[:, :_NS]
    v = [x[:, _NS + c * _NV:_NS + (c + 1) * _NV] for c in range(3)]
    sh0 = sh[:, 0:1]
    s123 = [sh[:, c + 1:c + 2] for c in range(3)]
    o0e = jnp.concatenate(
        [in0e * sh0, v[0] * s123[0] + v[1] * s123[1] + v[2] * s123[2]], axis=1)
    outs = [((o0e @ r16_ref[...]) * tpw0) @ s16_ref[...]]
    for c in range(3):
        o1o_c = jnp.concatenate([in0e * s123[c], v[c] * sh0], axis=1)
        outs.append(((o1o_c @ r8_ref[...]) * tpw1) @ s8_ref[...])
    t = out_ref.shape[0]
    outs.append(jnp.ones((t, 1), jnp.float32))
    outs.append(jnp.zeros((t, _MSG_W - _DIN - 1), jnp.float32))
    out_ref[...] = jnp.concatenate(outs, axis=1)


def _edge_messages(x, edge_attr, edge_sh, W1, b1, W2, b2):
    grid = _EP // _TILE
    spec = lambda cols: pl.BlockSpec((_TILE, cols), lambda i: (i, 0))
    const = lambda shape: pl.BlockSpec(shape, lambda i: (0, 0))
    return pl.pallas_call(
        _edge_kernel,
        grid=(grid,),
        in_specs=[
            spec(128), spec(16), spec(4),
            const((16, 64)), const((1, 64)), const((64, _WNUM)),
            const((1, _WNUM)),
            const(_R16.shape), const(_S16.shape), const(_R8.shape),
            const(_S8.shape),
        ],
        out_specs=spec(_MSG_W),
        out_shape=jax.ShapeDtypeStruct((_EP, _MSG_W), jnp.float32),
        compiler_params=pltpu.CompilerParams(
            dimension_semantics=("parallel",)),
    )(x, edge_attr, edge_sh, W1, b1.reshape(1, -1), W2, b2.reshape(1, -1),
      jnp.asarray(_R16), jnp.asarray(_S16), jnp.asarray(_R8), jnp.asarray(_S8))


def _gather_rows(table, dst_idx3):
    mesh = plsc.VectorSubcoreMesh(core_axis_name="c", subcore_axis_name="s")

    @functools.partial(
        pl.kernel, mesh=mesh,
        out_type=jax.ShapeDtypeStruct((_EP, 128), jnp.float32),
        scratch_types=[
            pltpu.VMEM((_NCHUNK, _CHUNK), jnp.int32),
            pltpu.VMEM((_CHUNK, 128), jnp.float32),
            pltpu.VMEM((_CHUNK, 128), jnp.float32),
            pltpu.SemaphoreType.DMA,
            pltpu.SemaphoreType.DMA,
            pltpu.SemaphoreType.DMA,
            pltpu.SemaphoreType.DMA,
        ],
    )
    def k(table_hbm, idx_hbm, out_hbm, idx_v, ra, rb, sga, sgb, swa, swb):
        c = lax.axis_index("c")
        s = lax.axis_index("s")
        w = c * 16 + s
        pltpu.sync_copy(idx_hbm.at[w], idx_v)

        def gather(j, buf, sem):
            pltpu.async_copy(table_hbm.at[idx_v.at[j]], buf, sem)

        def write(j, buf, sem):
            base = w * _EPW + j * _CHUNK
            pltpu.async_copy(buf, out_hbm.at[pl.ds(base, _CHUNK)], sem)

        def wait(src, dst, sem):
            pltpu.make_async_copy(src, dst, sem).wait()

        gather(0, ra, sga)

        def body(i, carry):
            j = 2 * i
            wait(table_hbm.at[idx_v.at[j]], ra, sga)
            gather(j + 1, rb, sgb)

            @pl.when(i > 0)
            def _():
                wait(ra, out_hbm.at[pl.ds(0, _CHUNK)], swa)
            write(j, ra, swa)
            wait(table_hbm.at[idx_v.at[j + 1]], rb, sgb)

            @pl.when(i + 1 < _NCHUNK // 2)
            def _():
                gather(j + 2, ra, sga)

            @pl.when(i > 0)
            def _():
                wait(rb, out_hbm.at[pl.ds(0, _CHUNK)], swb)
            write(j + 1, rb, swb)
            return carry

        lax.fori_loop(0, _NCHUNK // 2, body, 0)
        wait(ra, out_hbm.at[pl.ds(0, _CHUNK)], swa)
        wait(rb, out_hbm.at[pl.ds(0, _CHUNK)], swb)

    return k(table, dst_idx3)


def _scatter_partials(msg, src_idx3, zero_rows):
    mesh = plsc.VectorSubcoreMesh(core_axis_name="c", subcore_axis_name="s")

    @functools.partial(
        pl.kernel, mesh=mesh,
        out_type=jax.ShapeDtypeStruct((2, _NROWS, _MSG_W), jnp.float32),
        scratch_types=[
            pltpu.VMEM_SHARED((_NROWS, _MSG_W), jnp.float32),
            pltpu.VMEM((_NCHUNK, _CHUNK), jnp.int32),
            pltpu.VMEM((_CHUNK, _MSG_W), jnp.float32),
        ],
    )
    def k(msg_hbm, idx_hbm, zero_hbm, out_hbm, acc, idx_v, msg_v):
        c = lax.axis_index("c")
        s = lax.axis_index("s")
        w = c * 16 + s
        roff = s * _RPW
        pltpu.sync_copy(zero_hbm.at[pl.ds(roff, _RPW)], acc.at[pl.ds(roff, _RPW)])
        pltpu.sync_copy(idx_hbm.at[w], idx_v)
        plsc.subcore_barrier()

        def body(j, carry):
            base = w * _EPW + j * _CHUNK
            pltpu.sync_copy(msg_hbm.at[pl.ds(base, _CHUNK)], msg_v)
            pltpu.sync_copy(msg_v, acc.at[idx_v.at[j]], add=True)
            return carry

        lax.fori_loop(0, _NCHUNK, body, 0)
        plsc.subcore_barrier()
        pltpu.sync_copy(acc.at[pl.ds(roff, _RPW)],
                        out_hbm.at[c, pl.ds(roff, _RPW)])

    return k(msg, src_idx3, zero_rows)


def _epilogue_kernel(p_ref, na_ref, pinv_ref, out_ref):
    p = p_ref[0] + p_ref[1]
    cnt = p[:, _DIN:_DIN + 1]
    vals = p[:, :_DIN]
    out_ref[...] = (vals / jnp.maximum(cnt, 1.0)) @ pinv_ref[...] + na_ref[...]


def _epilogue(partials, node_attr, pinv):
    rows = 1256
    return pl.pallas_call(
        _epilogue_kernel,
        grid=(pl.cdiv(_NNODES, rows),),
        in_specs=[
            pl.BlockSpec((2, rows, _MSG_W), lambda i: (0, i, 0)),
            pl.BlockSpec((rows, _DIN), lambda i: (i, 0)),
            pl.BlockSpec((_DIN, _DIN), lambda i: (0, 0)),
        ],
        out_specs=pl.BlockSpec((rows, _DIN), lambda i: (i, 0)),
        out_shape=jax.ShapeDtypeStruct((_NNODES, _DIN), jnp.float32),
        compiler_params=pltpu.CompilerParams(
            dimension_semantics=("parallel",)),
    )(partials, node_attr, pinv)


def kernel(node_attr, edge_index, edge_attr, edge_sh, W1, b1, W2, b2):
    edge_src = edge_index[0]
    edge_dst = edge_index[1]
    pad = _EP - _E
    node_p = jnp.pad(node_attr[:, _PERM], ((0, 0), (0, 128 - _DIN)))
    dst_pad = jnp.concatenate([edge_dst, jnp.zeros((pad,), jnp.int32)])
    src_pad = jnp.concatenate(
        [edge_src, jnp.full((pad,), _NNODES, jnp.int32)])
    ea_pad = jnp.pad(edge_attr, ((0, pad), (0, 0)))
    sh_pad = jnp.pad(edge_sh, ((0, pad), (0, 0)))
    x = _gather_rows(node_p, dst_pad.reshape(_NW, _NCHUNK, _CHUNK))
    msg = _edge_messages(x, ea_pad, sh_pad, W1, b1, W2, b2)
    partials = _scatter_partials(
        msg, src_pad.reshape(_NW, _NCHUNK, _CHUNK),
        jnp.zeros((_NROWS, _MSG_W), jnp.float32))
    return _epilogue(partials, node_attr, jnp.asarray(_PINV))
